# baseline (speedup 1.0000x reference)
import functools

import jax
import jax.numpy as jnp
from jax import lax
from jax.experimental import pallas as pl
from jax.experimental.pallas import tpu as pltpu
from jax.experimental.pallas import tpu_sc as plsc

NN = 10000
NE = 320000
D = 128
NC, NS = 2, 16
NW = NC * NS
EPT = NE // NW
C = 125
NCH = EPT // C
NP = 10112
STRIPE = NP // NS


@functools.lru_cache(maxsize=None)
def _make_sc_agg(with_deg):
    mesh = plsc.VectorSubcoreMesh(core_axis_name="c", subcore_axis_name="s")
    nph = 4 if with_deg else 2
    PH = NCH // nph
    out_type = [jax.ShapeDtypeStruct((NC, NP, D), jnp.float32)]
    scratch = [
        pltpu.VMEM((PH, C), jnp.int32),
        pltpu.VMEM((PH, C), jnp.int32),
        pltpu.VMEM((C, D), jnp.float32),
        pltpu.VMEM((C, D), jnp.float32),
        pltpu.VMEM_SHARED((NP, D), jnp.float32),
        pltpu.SemaphoreType.DMA,
        pltpu.SemaphoreType.DMA,
    ]
    if with_deg:
        out_type.append(jax.ShapeDtypeStruct((NC, NP, 16), jnp.float32))
        scratch += [
            pltpu.VMEM((C, 16), jnp.float32),
            pltpu.VMEM_SHARED((NP, 16), jnp.float32),
        ]

    def body(*refs):
        if with_deg:
            (h_hbm, src_hbm, dst_hbm, z_hbm, z16_hbm, one16_hbm,
             acc_out, deg_out, sidx, didx, rows_a, rows_b, accsh,
             sem_a, sem_b, ones_v, degsh) = refs
        else:
            (h_hbm, src_hbm, dst_hbm, z_hbm, acc_out,
             sidx, didx, rows_a, rows_b, accsh, sem_a, sem_b) = refs
        ci = lax.axis_index("c")
        si = lax.axis_index("s")
        wid = ci * NS + si
        row0 = si * STRIPE
        pltpu.sync_copy(z_hbm.at[pl.ds(row0, STRIPE)],
                        accsh.at[pl.ds(row0, STRIPE)])
        if with_deg:
            pltpu.sync_copy(z16_hbm.at[pl.ds(row0, STRIPE)],
                            degsh.at[pl.ds(row0, STRIPE)])
            pltpu.sync_copy(one16_hbm, ones_v)
        plsc.subcore_barrier()

        for ph in range(nph):
            pltpu.sync_copy(src_hbm.at[wid, ph], sidx)
            pltpu.sync_copy(dst_hbm.at[wid, ph], didx)
            pltpu.async_copy(h_hbm.at[sidx.at[0]], rows_a, sem_a)

            @pl.loop(0, PH, step=2)
            def _(i):
                pltpu.async_copy(h_hbm.at[sidx.at[i + 1]], rows_b, sem_b)
                pltpu.make_async_copy(h_hbm.at[sidx.at[i]], rows_a, sem_a).wait()
                pltpu.sync_copy(rows_a, accsh.at[didx.at[i]], add=True)

                @pl.when(i < PH - 2)
                def _():
                    pltpu.async_copy(h_hbm.at[sidx.at[i + 2]], rows_a, sem_a)

                pltpu.make_async_copy(h_hbm.at[sidx.at[i + 1]], rows_b, sem_b).wait()
                pltpu.sync_copy(rows_b, accsh.at[didx.at[i + 1]], add=True)

                if with_deg:
                    pltpu.sync_copy(ones_v, degsh.at[didx.at[i]], add=True)
                    pltpu.sync_copy(ones_v, degsh.at[didx.at[i + 1]], add=True)

        plsc.subcore_barrier()
        pltpu.sync_copy(accsh.at[pl.ds(row0, STRIPE)],
                        acc_out.at[ci, pl.ds(row0, STRIPE)])
        if with_deg:
            pltpu.sync_copy(degsh.at[pl.ds(row0, STRIPE)],
                            deg_out.at[ci, pl.ds(row0, STRIPE)])

    return pl.kernel(
        body, out_type=out_type, mesh=mesh, scratch_types=scratch,
        compiler_params=pltpu.CompilerParams(use_tc_tiling_on_sc=False))


@functools.lru_cache(maxsize=None)
def _make_sc_agg3():
    mesh = plsc.VectorSubcoreMesh(core_axis_name="c", subcore_axis_name="s")
    nph = 10
    PH = NCH // nph
    out_type = [jax.ShapeDtypeStruct((NC, NP, D), jnp.float32)]
    scratch = [
        pltpu.VMEM((PH, C), jnp.int32),
        pltpu.VMEM((PH, C), jnp.int32),
        pltpu.VMEM((C, D), jnp.float32),
        pltpu.VMEM((C, D), jnp.float32),
        pltpu.VMEM((C, D), jnp.float32),
        pltpu.VMEM_SHARED((NP, D), jnp.float32),
        pltpu.SemaphoreType.DMA,
        pltpu.SemaphoreType.DMA,
        pltpu.SemaphoreType.DMA,
    ]

    def body(h_hbm, src_hbm, dst_hbm, z_hbm, acc_out,
             sidx, didx, r0, r1, r2, accsh, s0, s1, s2):
        rows = [r0, r1, r2]
        sems = [s0, s1, s2]
        ci = lax.axis_index("c")
        si = lax.axis_index("s")
        wid = ci * NS + si
        row0 = si * STRIPE
        pltpu.sync_copy(z_hbm.at[pl.ds(row0, STRIPE)],
                        accsh.at[pl.ds(row0, STRIPE)])
        plsc.subcore_barrier()

        for ph in range(nph):
            pltpu.sync_copy(src_hbm.at[wid, ph], sidx)
            pltpu.sync_copy(dst_hbm.at[wid, ph], didx)
            pltpu.async_copy(h_hbm.at[sidx.at[0]], rows[0], sems[0])
            pltpu.async_copy(h_hbm.at[sidx.at[1]], rows[1], sems[1])
            for j in range(PH):
                b = j % 3
                pltpu.make_async_copy(h_hbm.at[sidx.at[j]], rows[b],
                                      sems[b]).wait()
                if j + 2 < PH:
                    b2 = (j + 2) % 3
                    pltpu.async_copy(h_hbm.at[sidx.at[j + 2]], rows[b2],
                                     sems[b2])
                pltpu.sync_copy(rows[b], accsh.at[didx.at[j]], add=True)

        plsc.subcore_barrier()
        pltpu.sync_copy(accsh.at[pl.ds(row0, STRIPE)],
                        acc_out.at[ci, pl.ds(row0, STRIPE)])

    return pl.kernel(
        body, out_type=out_type, mesh=mesh, scratch_types=scratch,
        compiler_params=pltpu.CompilerParams(use_tc_tiling_on_sc=False))


@functools.lru_cache(maxsize=None)
def _make_sc_deg():
    mesh = plsc.VectorSubcoreMesh(core_axis_name="c", subcore_axis_name="s")
    out_type = [jax.ShapeDtypeStruct((NC, NP, 16), jnp.float32)]
    scratch = [
        pltpu.VMEM((NCH, C), jnp.int32),
        pltpu.VMEM((C, 16), jnp.float32),
        pltpu.VMEM_SHARED((NP, 16), jnp.float32),
    ]

    def body(dst_hbm, z16_hbm, one16_hbm, deg_out, didx, ones_v, degsh):
        ci = lax.axis_index("c")
        si = lax.axis_index("s")
        wid = ci * NS + si
        row0 = si * STRIPE
        pltpu.sync_copy(z16_hbm.at[pl.ds(row0, STRIPE)],
                        degsh.at[pl.ds(row0, STRIPE)])
        pltpu.sync_copy(dst_hbm.at[wid], didx)
        pltpu.sync_copy(one16_hbm, ones_v)
        plsc.subcore_barrier()

        @pl.loop(0, NCH)
        def _(j):
            pltpu.sync_copy(ones_v, degsh.at[didx.at[j]], add=True)

        plsc.subcore_barrier()
        pltpu.sync_copy(degsh.at[pl.ds(row0, STRIPE)],
                        deg_out.at[ci, pl.ds(row0, STRIPE)])

    return pl.kernel(
        body, out_type=out_type, mesh=mesh, scratch_types=scratch,
        compiler_params=pltpu.CompilerParams(use_tc_tiling_on_sc=False))


def _tc_layer(h, acc, deg16, wn, ws, bias):
    BLK = 2000
    G = NN // BLK

    def body(acc_ref, deg_ref, h_ref, wn_ref, ws_ref, b_ref, o_ref):
        a = acc_ref[0] + acc_ref[1]
        d = deg_ref[...]
        dg = d[0, :, 0:1] + d[1, :, 0:1]
        mean = a / jnp.maximum(dg, 1.0)
        out = (jnp.dot(mean, wn_ref[...], preferred_element_type=jnp.float32)
               + jnp.dot(h_ref[...], ws_ref[...], preferred_element_type=jnp.float32)
               + b_ref[...])
        out = jnp.maximum(out, 0.0)
        nrm = jnp.sqrt(jnp.sum(out * out, axis=1, keepdims=True))
        o_ref[...] = out / jnp.maximum(nrm, 1e-12)

    return pl.pallas_call(
        body,
        grid=(G,),
        in_specs=[
            pl.BlockSpec((NC, BLK, D), lambda i: (0, i, 0)),
            pl.BlockSpec((NC, BLK, 16), lambda i: (0, i, 0)),
            pl.BlockSpec((BLK, D), lambda i: (i, 0)),
            pl.BlockSpec((D, D), lambda i: (0, 0)),
            pl.BlockSpec((D, D), lambda i: (0, 0)),
            pl.BlockSpec((1, D), lambda i: (0, 0)),
        ],
        out_specs=pl.BlockSpec((BLK, D), lambda i: (i, 0)),
        out_shape=jax.ShapeDtypeStruct((NN, D), jnp.float32),
    )(acc, deg16, h, wn, ws, bias)


def _tc_layer_head(h, acc, deg16, wn, ws, bias, w1, b1, w2, b2):
    BLK = 2000
    G = NN // BLK

    def body(acc_ref, deg_ref, h_ref, wn_ref, ws_ref, b_ref,
             w1_ref, b1_ref, w2_ref, b2_ref, o_ref, sum_ref):
        i = pl.program_id(0)
        a = acc_ref[0] + acc_ref[1]
        d = deg_ref[...]
        dg = d[0, :, 0:1] + d[1, :, 0:1]
        mean = a / jnp.maximum(dg, 1.0)
        out = (jnp.dot(mean, wn_ref[...], preferred_element_type=jnp.float32)
               + jnp.dot(h_ref[...], ws_ref[...], preferred_element_type=jnp.float32)
               + b_ref[...])
        out = jnp.maximum(out, 0.0)
        nrm = jnp.sqrt(jnp.sum(out * out, axis=1, keepdims=True))
        out = out / jnp.maximum(nrm, 1e-12)
        part = jnp.sum(out, axis=0, keepdims=True)

        @pl.when(i == 0)
        def _():
            sum_ref[...] = jnp.zeros_like(sum_ref)

        sum_ref[...] += part

        @pl.when(i == G - 1)
        def _():
            g = sum_ref[...] * (1.0 / NN)
            g = jnp.maximum(
                jnp.dot(g, w1_ref[...], preferred_element_type=jnp.float32)
                + b1_ref[...], 0.0)
            o_ref[...] = (jnp.dot(g, w2_ref[...],
                                  preferred_element_type=jnp.float32)
                          + b2_ref[...])

    return pl.pallas_call(
        body,
        grid=(G,),
        in_specs=[
            pl.BlockSpec((NC, BLK, D), lambda i: (0, i, 0)),
            pl.BlockSpec((NC, BLK, 16), lambda i: (0, i, 0)),
            pl.BlockSpec((BLK, D), lambda i: (i, 0)),
            pl.BlockSpec((D, D), lambda i: (0, 0)),
            pl.BlockSpec((D, D), lambda i: (0, 0)),
            pl.BlockSpec((1, D), lambda i: (0, 0)),
            pl.BlockSpec((D, D // 2), lambda i: (0, 0)),
            pl.BlockSpec((1, D // 2), lambda i: (0, 0)),
            pl.BlockSpec((D // 2, 2), lambda i: (0, 0)),
            pl.BlockSpec((1, 2), lambda i: (0, 0)),
        ],
        out_specs=pl.BlockSpec((1, 2), lambda i: (0, 0)),
        out_shape=jax.ShapeDtypeStruct((1, 2), jnp.float32),
        scratch_shapes=[pltpu.VMEM((1, D), jnp.float32)],
    )(acc, deg16, h, wn, ws, bias, w1, b1, w2, b2)


def kernel(x, edge_index, W_neigh, W_self, b, Wc1, bc1, Wc2, bc2):
    src_i = edge_index[0].astype(jnp.int32)
    dst_i = edge_index[1].astype(jnp.int32)
    src = src_i.reshape(NW, 10, NCH // 10, C)
    dst = dst_i.reshape(NW, 10, NCH // 10, C)
    dst3 = dst_i.reshape(NW, NCH, C)
    zeros = jnp.zeros((NP, D), jnp.float32)
    zeros16 = jnp.zeros((NP, 16), jnp.float32)
    ones16 = jnp.ones((C, 16), jnp.float32)

    h = x
    (deg16,) = _make_sc_deg()(dst3, zeros16, ones16)
    for l in range(2):
        (acc,) = _make_sc_agg3()(h, src, dst, zeros)
        h = _tc_layer(h, acc, deg16, W_neigh[l], W_self[l], b[l].reshape(1, D))
    (acc,) = _make_sc_agg3()(h, src, dst, zeros)
    return _tc_layer_head(h, acc, deg16, W_neigh[2], W_self[2],
                          b[2].reshape(1, D), Wc1, bc1.reshape(1, -1),
                          Wc2, bc2.reshape(1, -1))


# speedup vs baseline: 1.0761x; 1.0761x over previous
import functools

import jax
import jax.numpy as jnp
from jax import lax
from jax.experimental import pallas as pl
from jax.experimental.pallas import tpu as pltpu
from jax.experimental.pallas import tpu_sc as plsc

NN = 10000
NE = 320000
D = 128
NC, NS = 2, 16
NW = NC * NS
EPT = NE // NW
C = 125
NCH = EPT // C
NP = 10112
STRIPE = NP // NS


H2 = NCH // 2


@functools.lru_cache(maxsize=None)
def _make_sc_agg():
    mesh = plsc.VectorSubcoreMesh(core_axis_name="c", subcore_axis_name="s")
    out_type = [jax.ShapeDtypeStruct((NC, NP, D), jnp.float32)]
    scratch = [
        pltpu.VMEM((H2, C), jnp.int32),
        pltpu.VMEM((H2, C), jnp.int32),
        pltpu.VMEM((C, D), jnp.float32),
        pltpu.VMEM((C, D), jnp.float32),
        pltpu.VMEM_SHARED((NP, D), jnp.float32),
        pltpu.SemaphoreType.DMA,
        pltpu.SemaphoreType.DMA,
    ]

    def body(h_hbm, src_hbm, dst_hbm, z_hbm, acc_out,
             sidx, didx, rows_a, rows_b, accsh, sem_a, sem_b):
        ci = lax.axis_index("c")
        si = lax.axis_index("s")
        wid = ci * NS + si
        row0 = si * STRIPE
        pltpu.sync_copy(z_hbm.at[pl.ds(row0, STRIPE)],
                        accsh.at[pl.ds(row0, STRIPE)])
        plsc.subcore_barrier()

        for half in range(2):
            pltpu.sync_copy(src_hbm.at[wid, half], sidx)
            pltpu.sync_copy(dst_hbm.at[wid, half], didx)
            pltpu.async_copy(h_hbm.at[sidx.at[0]], rows_a, sem_a)

            @pl.loop(0, H2, step=2)
            def _(i):
                pltpu.async_copy(h_hbm.at[sidx.at[i + 1]], rows_b, sem_b)
                pltpu.make_async_copy(h_hbm.at[sidx.at[i]], rows_a, sem_a).wait()
                pltpu.sync_copy(rows_a, accsh.at[didx.at[i]], add=True)

                @pl.when(i < H2 - 2)
                def _():
                    pltpu.async_copy(h_hbm.at[sidx.at[i + 2]], rows_a, sem_a)

                pltpu.make_async_copy(h_hbm.at[sidx.at[i + 1]], rows_b, sem_b).wait()
                pltpu.sync_copy(rows_b, accsh.at[didx.at[i + 1]], add=True)

        plsc.subcore_barrier()
        pltpu.sync_copy(accsh.at[pl.ds(row0, STRIPE)],
                        acc_out.at[ci, pl.ds(row0, STRIPE)])

    return pl.kernel(
        body, out_type=out_type, mesh=mesh, scratch_types=scratch,
        compiler_params=pltpu.CompilerParams(use_tc_tiling_on_sc=False))


@functools.lru_cache(maxsize=None)
def _make_sc_deg():
    mesh = plsc.VectorSubcoreMesh(core_axis_name="c", subcore_axis_name="s")
    out_type = [jax.ShapeDtypeStruct((NC, NP, 16), jnp.float32)]
    scratch = [
        pltpu.VMEM((NCH, C), jnp.int32),
        pltpu.VMEM((C, 16), jnp.float32),
        pltpu.VMEM_SHARED((NP, 16), jnp.float32),
    ]

    def body(dst_hbm, z16_hbm, one16_hbm, deg_out, didx, ones_v, degsh):
        ci = lax.axis_index("c")
        si = lax.axis_index("s")
        wid = ci * NS + si
        row0 = si * STRIPE
        pltpu.sync_copy(z16_hbm.at[pl.ds(row0, STRIPE)],
                        degsh.at[pl.ds(row0, STRIPE)])
        pltpu.sync_copy(dst_hbm.at[wid], didx)
        pltpu.sync_copy(one16_hbm, ones_v)
        plsc.subcore_barrier()

        @pl.loop(0, NCH)
        def _(j):
            pltpu.sync_copy(ones_v, degsh.at[didx.at[j]], add=True)

        plsc.subcore_barrier()
        pltpu.sync_copy(degsh.at[pl.ds(row0, STRIPE)],
                        deg_out.at[ci, pl.ds(row0, STRIPE)])

    return pl.kernel(
        body, out_type=out_type, mesh=mesh, scratch_types=scratch,
        compiler_params=pltpu.CompilerParams(use_tc_tiling_on_sc=False))


def _tc_layer(h, acc, deg16, wn, ws, bias):
    BLK = 2000
    G = NN // BLK

    def body(acc_ref, deg_ref, h_ref, wn_ref, ws_ref, b_ref, o_ref):
        a = acc_ref[0] + acc_ref[1]
        d = deg_ref[...]
        dg = d[0, :, 0:1] + d[1, :, 0:1]
        mean = a / jnp.maximum(dg, 1.0)
        out = (jnp.dot(mean, wn_ref[...], preferred_element_type=jnp.float32)
               + jnp.dot(h_ref[...], ws_ref[...], preferred_element_type=jnp.float32)
               + b_ref[...])
        out = jnp.maximum(out, 0.0)
        nrm = jnp.sqrt(jnp.sum(out * out, axis=1, keepdims=True))
        o_ref[...] = out / jnp.maximum(nrm, 1e-12)

    return pl.pallas_call(
        body,
        grid=(G,),
        in_specs=[
            pl.BlockSpec((NC, BLK, D), lambda i: (0, i, 0)),
            pl.BlockSpec((NC, BLK, 16), lambda i: (0, i, 0)),
            pl.BlockSpec((BLK, D), lambda i: (i, 0)),
            pl.BlockSpec((D, D), lambda i: (0, 0)),
            pl.BlockSpec((D, D), lambda i: (0, 0)),
            pl.BlockSpec((1, D), lambda i: (0, 0)),
        ],
        out_specs=pl.BlockSpec((BLK, D), lambda i: (i, 0)),
        out_shape=jax.ShapeDtypeStruct((NN, D), jnp.float32),
    )(acc, deg16, h, wn, ws, bias)


def _tc_layer_head(h, acc, deg16, wn, ws, bias, w1, b1, w2, b2):
    BLK = 2000
    G = NN // BLK

    def body(acc_ref, deg_ref, h_ref, wn_ref, ws_ref, b_ref,
             w1_ref, b1_ref, w2_ref, b2_ref, o_ref, sum_ref):
        i = pl.program_id(0)
        a = acc_ref[0] + acc_ref[1]
        d = deg_ref[...]
        dg = d[0, :, 0:1] + d[1, :, 0:1]
        mean = a / jnp.maximum(dg, 1.0)
        out = (jnp.dot(mean, wn_ref[...], preferred_element_type=jnp.float32)
               + jnp.dot(h_ref[...], ws_ref[...], preferred_element_type=jnp.float32)
               + b_ref[...])
        out = jnp.maximum(out, 0.0)
        nrm = jnp.sqrt(jnp.sum(out * out, axis=1, keepdims=True))
        out = out / jnp.maximum(nrm, 1e-12)
        part = jnp.sum(out, axis=0, keepdims=True)

        @pl.when(i == 0)
        def _():
            sum_ref[...] = jnp.zeros_like(sum_ref)

        sum_ref[...] += part

        @pl.when(i == G - 1)
        def _():
            g = sum_ref[...] * (1.0 / NN)
            g = jnp.maximum(
                jnp.dot(g, w1_ref[...], preferred_element_type=jnp.float32)
                + b1_ref[...], 0.0)
            o_ref[...] = (jnp.dot(g, w2_ref[...],
                                  preferred_element_type=jnp.float32)
                          + b2_ref[...])

    return pl.pallas_call(
        body,
        grid=(G,),
        in_specs=[
            pl.BlockSpec((NC, BLK, D), lambda i: (0, i, 0)),
            pl.BlockSpec((NC, BLK, 16), lambda i: (0, i, 0)),
            pl.BlockSpec((BLK, D), lambda i: (i, 0)),
            pl.BlockSpec((D, D), lambda i: (0, 0)),
            pl.BlockSpec((D, D), lambda i: (0, 0)),
            pl.BlockSpec((1, D), lambda i: (0, 0)),
            pl.BlockSpec((D, D // 2), lambda i: (0, 0)),
            pl.BlockSpec((1, D // 2), lambda i: (0, 0)),
            pl.BlockSpec((D // 2, 2), lambda i: (0, 0)),
            pl.BlockSpec((1, 2), lambda i: (0, 0)),
        ],
        out_specs=pl.BlockSpec((1, 2), lambda i: (0, 0)),
        out_shape=jax.ShapeDtypeStruct((1, 2), jnp.float32),
        scratch_shapes=[pltpu.VMEM((1, D), jnp.float32)],
    )(acc, deg16, h, wn, ws, bias, w1, b1, w2, b2)


def kernel(x, edge_index, W_neigh, W_self, b, Wc1, bc1, Wc2, bc2):
    src_i = edge_index[0].astype(jnp.int32)
    dst_i = edge_index[1].astype(jnp.int32)
    src = src_i.reshape(NW, 2, NCH // 2, C)
    dst = dst_i.reshape(NW, 2, NCH // 2, C)
    dst3 = dst_i.reshape(NW, NCH, C)
    zeros = jnp.zeros((NP, D), jnp.float32)
    zeros16 = jnp.zeros((NP, 16), jnp.float32)
    ones16 = jnp.ones((C, 16), jnp.float32)

    h = x
    (deg16,) = _make_sc_deg()(dst3, zeros16, ones16)
    for l in range(2):
        (acc,) = _make_sc_agg()(h, src, dst, zeros)
        h = _tc_layer(h, acc, deg16, W_neigh[l], W_self[l], b[l].reshape(1, D))
    (acc,) = _make_sc_agg()(h, src, dst, zeros)
    return _tc_layer_head(h, acc, deg16, W_neigh[2], W_self[2],
                          b[2].reshape(1, D), Wc1, bc1.reshape(1, -1),
                          Wc2, bc2.reshape(1, -1))


# speedup vs baseline: 1.0804x; 1.0040x over previous
import functools

import jax
import jax.numpy as jnp
from jax import lax
from jax.experimental import pallas as pl
from jax.experimental.pallas import tpu as pltpu
from jax.experimental.pallas import tpu_sc as plsc

NN = 10000
NE = 320000
D = 128
NC, NS = 2, 16
NW = NC * NS
EPT = NE // NW
C = 125
NCH = EPT // C
NP = 10112
STRIPE = NP // NS


H2 = NCH // 2


@functools.lru_cache(maxsize=None)
def _make_sc_agg():
    mesh = plsc.VectorSubcoreMesh(core_axis_name="c", subcore_axis_name="s")
    out_type = [jax.ShapeDtypeStruct((NC, NP, D), jnp.float32)]
    scratch = [
        pltpu.VMEM((H2, C), jnp.int32),
        pltpu.VMEM((H2, C), jnp.int32),
        pltpu.VMEM((C, D), jnp.float32),
        pltpu.VMEM((C, D), jnp.float32),
        pltpu.VMEM_SHARED((NP, D), jnp.float32),
        pltpu.SemaphoreType.DMA,
        pltpu.SemaphoreType.DMA,
    ]

    def body(h_hbm, src_hbm, dst_hbm, z_hbm, acc_out,
             sidx, didx, rows_a, rows_b, accsh, sem_a, sem_b):
        ci = lax.axis_index("c")
        si = lax.axis_index("s")
        wid = ci * NS + si
        row0 = si * STRIPE
        pltpu.sync_copy(z_hbm.at[pl.ds(row0, STRIPE)],
                        accsh.at[pl.ds(row0, STRIPE)])
        plsc.subcore_barrier()

        for half in range(2):
            pltpu.sync_copy(src_hbm.at[wid, half], sidx)
            pltpu.sync_copy(dst_hbm.at[wid, half], didx)
            pltpu.async_copy(h_hbm.at[sidx.at[0]], rows_a, sem_a)

            @pl.loop(0, H2, step=2)
            def _(i):
                pltpu.async_copy(h_hbm.at[sidx.at[i + 1]], rows_b, sem_b)
                pltpu.make_async_copy(h_hbm.at[sidx.at[i]], rows_a, sem_a).wait()
                pltpu.sync_copy(rows_a, accsh.at[didx.at[i]], add=True)

                @pl.when(i < H2 - 2)
                def _():
                    pltpu.async_copy(h_hbm.at[sidx.at[i + 2]], rows_a, sem_a)

                pltpu.make_async_copy(h_hbm.at[sidx.at[i + 1]], rows_b, sem_b).wait()
                pltpu.sync_copy(rows_b, accsh.at[didx.at[i + 1]], add=True)

        plsc.subcore_barrier()
        pltpu.sync_copy(accsh.at[pl.ds(row0, STRIPE)],
                        acc_out.at[ci, pl.ds(row0, STRIPE)])

    return pl.kernel(
        body, out_type=out_type, mesh=mesh, scratch_types=scratch,
        compiler_params=pltpu.CompilerParams(use_tc_tiling_on_sc=False))


@functools.lru_cache(maxsize=None)
def _make_sc_deg():
    mesh = plsc.VectorSubcoreMesh(core_axis_name="c", subcore_axis_name="s")
    out_type = [jax.ShapeDtypeStruct((NC, NP, 16), jnp.float32)]
    scratch = [
        pltpu.VMEM((NCH, C), jnp.int32),
        pltpu.VMEM((C, 16), jnp.float32),
        pltpu.VMEM_SHARED((NP, 16), jnp.float32),
    ]

    def body(dst_hbm, z16_hbm, one16_hbm, deg_out, didx, ones_v, degsh):
        ci = lax.axis_index("c")
        si = lax.axis_index("s")
        wid = ci * NS + si
        row0 = si * STRIPE
        pltpu.sync_copy(z16_hbm.at[pl.ds(row0, STRIPE)],
                        degsh.at[pl.ds(row0, STRIPE)])
        pltpu.sync_copy(dst_hbm.at[wid], didx)
        pltpu.sync_copy(one16_hbm, ones_v)
        plsc.subcore_barrier()

        @pl.loop(0, NCH)
        def _(j):
            pltpu.sync_copy(ones_v, degsh.at[didx.at[j]], add=True)

        plsc.subcore_barrier()
        pltpu.sync_copy(degsh.at[pl.ds(row0, STRIPE)],
                        deg_out.at[ci, pl.ds(row0, STRIPE)])

    return pl.kernel(
        body, out_type=out_type, mesh=mesh, scratch_types=scratch,
        compiler_params=pltpu.CompilerParams(use_tc_tiling_on_sc=False))


def _tc_layer(h, acc, deg16, wn, ws, bias):
    BLK = 5000
    G = NN // BLK

    def body(acc_ref, deg_ref, h_ref, wn_ref, ws_ref, b_ref, o_ref):
        a = acc_ref[0] + acc_ref[1]
        d = deg_ref[...]
        dg = d[0, :, 0:1] + d[1, :, 0:1]
        mean = a / jnp.maximum(dg, 1.0)
        out = (jnp.dot(mean, wn_ref[...], preferred_element_type=jnp.float32)
               + jnp.dot(h_ref[...], ws_ref[...], preferred_element_type=jnp.float32)
               + b_ref[...])
        out = jnp.maximum(out, 0.0)
        nrm = jnp.sqrt(jnp.sum(out * out, axis=1, keepdims=True))
        o_ref[...] = out / jnp.maximum(nrm, 1e-12)

    return pl.pallas_call(
        body,
        grid=(G,),
        in_specs=[
            pl.BlockSpec((NC, BLK, D), lambda i: (0, i, 0)),
            pl.BlockSpec((NC, BLK, 16), lambda i: (0, i, 0)),
            pl.BlockSpec((BLK, D), lambda i: (i, 0)),
            pl.BlockSpec((D, D), lambda i: (0, 0)),
            pl.BlockSpec((D, D), lambda i: (0, 0)),
            pl.BlockSpec((1, D), lambda i: (0, 0)),
        ],
        out_specs=pl.BlockSpec((BLK, D), lambda i: (i, 0)),
        out_shape=jax.ShapeDtypeStruct((NN, D), jnp.float32),
    )(acc, deg16, h, wn, ws, bias)


def _tc_layer_head(h, acc, deg16, wn, ws, bias, w1, b1, w2, b2):
    BLK = 5000
    G = NN // BLK

    def body(acc_ref, deg_ref, h_ref, wn_ref, ws_ref, b_ref,
             w1_ref, b1_ref, w2_ref, b2_ref, o_ref, sum_ref):
        i = pl.program_id(0)
        a = acc_ref[0] + acc_ref[1]
        d = deg_ref[...]
        dg = d[0, :, 0:1] + d[1, :, 0:1]
        mean = a / jnp.maximum(dg, 1.0)
        out = (jnp.dot(mean, wn_ref[...], preferred_element_type=jnp.float32)
               + jnp.dot(h_ref[...], ws_ref[...], preferred_element_type=jnp.float32)
               + b_ref[...])
        out = jnp.maximum(out, 0.0)
        nrm = jnp.sqrt(jnp.sum(out * out, axis=1, keepdims=True))
        out = out / jnp.maximum(nrm, 1e-12)
        part = jnp.sum(out, axis=0, keepdims=True)

        @pl.when(i == 0)
        def _():
            sum_ref[...] = jnp.zeros_like(sum_ref)

        sum_ref[...] += part

        @pl.when(i == G - 1)
        def _():
            g = sum_ref[...] * (1.0 / NN)
            g = jnp.maximum(
                jnp.dot(g, w1_ref[...], preferred_element_type=jnp.float32)
                + b1_ref[...], 0.0)
            o_ref[...] = (jnp.dot(g, w2_ref[...],
                                  preferred_element_type=jnp.float32)
                          + b2_ref[...])

    return pl.pallas_call(
        body,
        grid=(G,),
        in_specs=[
            pl.BlockSpec((NC, BLK, D), lambda i: (0, i, 0)),
            pl.BlockSpec((NC, BLK, 16), lambda i: (0, i, 0)),
            pl.BlockSpec((BLK, D), lambda i: (i, 0)),
            pl.BlockSpec((D, D), lambda i: (0, 0)),
            pl.BlockSpec((D, D), lambda i: (0, 0)),
            pl.BlockSpec((1, D), lambda i: (0, 0)),
            pl.BlockSpec((D, D // 2), lambda i: (0, 0)),
            pl.BlockSpec((1, D // 2), lambda i: (0, 0)),
            pl.BlockSpec((D // 2, 2), lambda i: (0, 0)),
            pl.BlockSpec((1, 2), lambda i: (0, 0)),
        ],
        out_specs=pl.BlockSpec((1, 2), lambda i: (0, 0)),
        out_shape=jax.ShapeDtypeStruct((1, 2), jnp.float32),
        scratch_shapes=[pltpu.VMEM((1, D), jnp.float32)],
    )(acc, deg16, h, wn, ws, bias, w1, b1, w2, b2)


def kernel(x, edge_index, W_neigh, W_self, b, Wc1, bc1, Wc2, bc2):
    src_i = edge_index[0].astype(jnp.int32)
    dst_i = edge_index[1].astype(jnp.int32)
    src = src_i.reshape(NW, 2, NCH // 2, C)
    dst = dst_i.reshape(NW, 2, NCH // 2, C)
    dst3 = dst_i.reshape(NW, NCH, C)
    zeros = jnp.zeros((NP, D), jnp.float32)
    zeros16 = jnp.zeros((NP, 16), jnp.float32)
    ones16 = jnp.ones((C, 16), jnp.float32)

    h = x
    (deg16,) = _make_sc_deg()(dst3, zeros16, ones16)
    for l in range(2):
        (acc,) = _make_sc_agg()(h, src, dst, zeros)
        h = _tc_layer(h, acc, deg16, W_neigh[l], W_self[l], b[l].reshape(1, D))
    (acc,) = _make_sc_agg()(h, src, dst, zeros)
    return _tc_layer_head(h, acc, deg16, W_neigh[2], W_self[2],
                          b[2].reshape(1, D), Wc1, bc1.reshape(1, -1),
                          Wc2, bc2.reshape(1, -1))
